# baseline (speedup 1.0000x reference)
import jax
import jax.numpy as jnp
from jax.experimental import pallas as pl
from jax.experimental.pallas import tpu as pltpu


def _dma_copy(x_hbm, o_hbm, vmem, sem_in, sem_out):
    pltpu.make_async_copy(x_hbm, vmem, sem_in).start()
    pltpu.make_async_copy(x_hbm, vmem, sem_in).wait()
    pltpu.make_async_copy(vmem, o_hbm, sem_out).start()
    pltpu.make_async_copy(vmem, o_hbm, sem_out).wait()


def kernel(x):
    B, N, C = x.shape
    total = B * N * C
    xf = x.reshape(total)
    out = pl.pallas_call(
        _dma_copy,
        in_specs=[pl.BlockSpec(memory_space=pl.ANY)],
        out_specs=pl.BlockSpec(memory_space=pl.ANY),
        scratch_shapes=[
            pltpu.VMEM((total,), x.dtype),
            pltpu.SemaphoreType.DMA,
            pltpu.SemaphoreType.DMA,
        ],
        out_shape=jax.ShapeDtypeStruct((total,), x.dtype),
    )(xf)
    return out.reshape(B, N, C)


# speedup vs baseline: 1.8532x; 1.8532x over previous
import jax
import jax.numpy as jnp
from jax.experimental import pallas as pl
from jax.experimental.pallas import tpu as pltpu


def _dma_copy(x_hbm, o_hbm, vmem, sem_in, sem_out):
    pltpu.make_async_copy(x_hbm, vmem, sem_in).start()
    pltpu.make_async_copy(x_hbm, vmem, sem_in).wait()
    pltpu.make_async_copy(vmem, o_hbm, sem_out).start()
    pltpu.make_async_copy(vmem, o_hbm, sem_out).wait()


def kernel(x):
    return pl.pallas_call(
        _dma_copy,
        in_specs=[pl.BlockSpec(memory_space=pl.ANY)],
        out_specs=pl.BlockSpec(memory_space=pl.ANY),
        scratch_shapes=[
            pltpu.VMEM(x.shape, x.dtype),
            pltpu.SemaphoreType.DMA,
            pltpu.SemaphoreType.DMA,
        ],
        out_shape=jax.ShapeDtypeStruct(x.shape, x.dtype),
    )(x)


# speedup vs baseline: 10.8492x; 5.8541x over previous
import jax
import jax.numpy as jnp
from jax.experimental import pallas as pl
from jax.experimental.pallas import tpu as pltpu


def _dma_copy(x_hbm, o_hbm, vmem, sem_in, sem_out):
    pltpu.make_async_copy(x_hbm, vmem, sem_in).start()
    pltpu.make_async_copy(x_hbm, vmem, sem_in).wait()
    pltpu.make_async_copy(vmem, o_hbm, sem_out).start()
    pltpu.make_async_copy(vmem, o_hbm, sem_out).wait()


def kernel(x):
    B, N, C = x.shape
    J = N // 128
    M = B * J
    y = x.reshape(B, J, 128, C).transpose(3, 1, 0, 2).reshape(C, M, 128)
    o = pl.pallas_call(
        _dma_copy,
        in_specs=[pl.BlockSpec(memory_space=pl.ANY)],
        out_specs=pl.BlockSpec(memory_space=pl.ANY),
        scratch_shapes=[
            pltpu.VMEM((C, M, 128), x.dtype),
            pltpu.SemaphoreType.DMA,
            pltpu.SemaphoreType.DMA,
        ],
        out_shape=jax.ShapeDtypeStruct((C, M, 128), x.dtype),
    )(y)
    return o.reshape(C, J, B, 128).transpose(2, 1, 3, 0).reshape(B, N, C)


# speedup vs baseline: 12.9764x; 1.1961x over previous
import jax
import jax.numpy as jnp
from jax.experimental import pallas as pl
from jax.experimental.pallas import tpu as pltpu

_NCHUNKS = 4


def _chunks(total):
    base = total // _NCHUNKS
    rem = total % _NCHUNKS
    sizes = [base + (1 if i < rem else 0) for i in range(_NCHUNKS)]
    starts = [sum(sizes[:i]) for i in range(_NCHUNKS)]
    return list(zip(starts, sizes))


def _dma_copy(x_hbm, o_hbm, vmem, sem_in, sem_out):
    spans = _chunks(x_hbm.shape[0])
    for i, (s, n) in enumerate(spans):
        pltpu.make_async_copy(
            x_hbm.at[pl.ds(s, n)], vmem.at[pl.ds(s, n)], sem_in.at[i]
        ).start()
    for i, (s, n) in enumerate(spans):
        pltpu.make_async_copy(
            x_hbm.at[pl.ds(s, n)], vmem.at[pl.ds(s, n)], sem_in.at[i]
        ).wait()
        pltpu.make_async_copy(
            vmem.at[pl.ds(s, n)], o_hbm.at[pl.ds(s, n)], sem_out.at[i]
        ).start()
    for i, (s, n) in enumerate(spans):
        pltpu.make_async_copy(
            vmem.at[pl.ds(s, n)], o_hbm.at[pl.ds(s, n)], sem_out.at[i]
        ).wait()


def kernel(x):
    B, N, C = x.shape
    J = N // 128
    M = B * J
    y = x.reshape(B, J, 128, C).transpose(3, 1, 0, 2).reshape(C, M, 128)
    o = pl.pallas_call(
        _dma_copy,
        in_specs=[pl.BlockSpec(memory_space=pl.ANY)],
        out_specs=pl.BlockSpec(memory_space=pl.ANY),
        scratch_shapes=[
            pltpu.VMEM((C, M, 128), x.dtype),
            pltpu.SemaphoreType.DMA((_NCHUNKS,)),
            pltpu.SemaphoreType.DMA((_NCHUNKS,)),
        ],
        out_shape=jax.ShapeDtypeStruct((C, M, 128), x.dtype),
    )(y)
    return o.reshape(C, J, B, 128).transpose(2, 1, 3, 0).reshape(B, N, C)
